# baseline (speedup 1.0000x reference)
import jax
import jax.numpy as jnp
from jax import lax
from jax.experimental import pallas as pl
from jax.experimental.pallas import tpu as pltpu
from jax.experimental.pallas import tpu_sc as plsc

N = 10000
E = 320000
D = 128
DE = 16
H = 128

NC = 2
NS = 16
L = 16

CK = 64
NCH = 320
NCH2 = NCH // NC
EP = CK * NCH * NS
G = 8
NG1 = NCH // G
NG2 = NCH2 // G
NB = 4
RPT = 624
REM = N - RPT * NS
EPR = 128
QR = H // L

_SC_PARAMS = pltpu.CompilerParams(needs_layout_passes=False,
                                  use_tc_tiling_on_sc=False)
_MESH = plsc.VectorSubcoreMesh(core_axis_name="c", subcore_axis_name="s")


def _pre_body(x_ref, ea_ref, w_ref, asr_ref, adr_ref, wer_ref, aer_ref,
              h_ref, as_ref, ad_ref, ae_ref, sh_ref):
    h = jnp.dot(x_ref[...], w_ref[...], preferred_element_type=jnp.float32)
    h_ref[...] = h
    a_s = jnp.dot(h, asr_ref[...], preferred_element_type=jnp.float32)
    a_d = jnp.dot(h, adr_ref[...], preferred_element_type=jnp.float32)
    as_ref[...] = a_s
    ad_ref[...] = a_d
    u = jnp.dot(wer_ref[...], aer_ref[...], preferred_element_type=jnp.float32)
    urep = jnp.concatenate([u] * EPR, axis=0)
    row = lax.broadcasted_iota(jnp.int32, (DE * EPR, EPR), 0)
    col = lax.broadcasted_iota(jnp.int32, (DE * EPR, EPR), 1)
    u3 = jnp.where((row // DE) == col, urep, 0.0)
    ae = jnp.dot(ea_ref[...], u3, preferred_element_type=jnp.float32)
    ae_ref[...] = ae
    sh = jnp.maximum(jnp.max(a_s) + jnp.max(a_d) + jnp.max(ae), 0.0)
    sh_ref[...] = jnp.zeros((1, 1), jnp.float32) + sh


_pre = pl.pallas_call(
    _pre_body,
    out_shape=[
        jax.ShapeDtypeStruct((N, H), jnp.float32),
        jax.ShapeDtypeStruct((N, 1), jnp.float32),
        jax.ShapeDtypeStruct((N, 1), jnp.float32),
        jax.ShapeDtypeStruct((E // EPR, EPR), jnp.float32),
        jax.ShapeDtypeStruct((1, 1), jnp.float32),
    ],
)


def _sca_body(pk_h, as_h, ad_h, sh_h, ex_h, den_h,
              asv, adv, dnv, shv, pkb, exg, dstg, exo, sden_a,
              semsa, semsb):
    c = lax.axis_index("c")
    s = lax.axis_index("s")
    zero = jnp.zeros((L,), jnp.float32)
    izero = jnp.zeros((L,), jnp.int32)

    pltpu.sync_copy(as_h, asv)
    pltpu.sync_copy(ad_h, adv)
    pltpu.sync_copy(sh_h, shv)

    for k in range(2 * CK // L):
        sl = pl.ds(k * L, L)
        for p in range(2):
            exg[p, sl] = zero
            dstg[p, sl] = (lax.iota(jnp.int32, L)
                           + (s * 4 * CK + p * 2 * CK + k * L))

    def zden(i, carry):
        dnv[pl.ds(i * L, L)] = zero
        return carry
    lax.fori_loop(0, N // L, zden, 0)

    @pl.when(s == 0)
    def _():
        pltpu.sync_copy(dnv, sden_a)

    plsc.subcore_barrier()

    shift = shv[...]

    pltpu.async_copy(exg.at[0], sden_a.at[dstg.at[0]], semsa, add=True)
    pltpu.async_copy(exg.at[1], sden_a.at[dstg.at[1]], semsb, add=True)

    def p1(g, carry):
        pltpu.sync_copy(pk_h.at[pl.ds(s * NCH + g * G, G)], pkb)
        for pair in range(G // 2):
            p = pair % 2
            sem = semsa if p == 0 else semsb
            pltpu.make_async_copy(exg.at[p], sden_a.at[dstg.at[p]], sem).wait()
            for u in range(2):
                tc = pair * 2 + u
                for k in range(CK // L):
                    sl = pl.ds(k * L, L)
                    sv = pkb[tc, 0, sl]
                    dv = pkb[tc, 1, sl]
                    ae = plsc.bitcast(pkb[tc, 2, sl], jnp.float32)
                    av = plsc.load_gather(asv, [sv])
                    bv = plsc.load_gather(adv, [dv])
                    al = av + bv + ae
                    al = jnp.where(al >= 0.0, al, al * 0.2)
                    ex = jnp.exp(al - shift)
                    exg[p, pl.ds(u * CK + k * L, L)] = ex
                    dstg[p, pl.ds(u * CK + k * L, L)] = dv
                    exo[pl.ds(tc * CK + k * L, L)] = ex
            pltpu.async_copy(exg.at[p], sden_a.at[dstg.at[p]], sem, add=True)
        pltpu.sync_copy(exo, ex_h.at[pl.ds((s * NCH + g * G) * CK, G * CK)])
        return carry
    lax.fori_loop(0, NG1, p1, 0)
    pltpu.make_async_copy(exg.at[0], sden_a.at[dstg.at[0]], semsa).wait()
    pltpu.make_async_copy(exg.at[1], sden_a.at[dstg.at[1]], semsb).wait()

    plsc.subcore_barrier()

    base = s * RPT
    pltpu.sync_copy(sden_a.at[pl.ds(base, RPT)], dnv.at[pl.ds(0, RPT)])
    pltpu.sync_copy(dnv.at[pl.ds(0, RPT)], den_h.at[c, pl.ds(base, RPT)])

    @pl.when(s == 0)
    def _():
        pltpu.sync_copy(sden_a.at[pl.ds(RPT * NS, REM)],
                        dnv.at[pl.ds(RPT, REM)])
        pltpu.sync_copy(dnv.at[pl.ds(RPT, REM)],
                        den_h.at[c, pl.ds(RPT * NS, REM)])


_sca = pl.kernel(
    _sca_body,
    out_type=[
        jax.ShapeDtypeStruct((EP,), jnp.float32),
        jax.ShapeDtypeStruct((NC, N), jnp.float32),
    ],
    mesh=_MESH,
    compiler_params=_SC_PARAMS,
    scratch_types=[
        pltpu.VMEM((N,), jnp.float32),
        pltpu.VMEM((N,), jnp.float32),
        pltpu.VMEM((N,), jnp.float32),
        pltpu.VMEM((L,), jnp.float32),
        pltpu.VMEM((G, 3, CK), jnp.int32),
        pltpu.VMEM((2, 2 * CK), jnp.float32),
        pltpu.VMEM((2, 2 * CK), jnp.int32),
        pltpu.VMEM((G * CK,), jnp.float32),
        pltpu.VMEM_SHARED((N,), jnp.float32),
        pltpu.SemaphoreType.DMA,
        pltpu.SemaphoreType.DMA,
    ],
)


def _scb_body(pk_h, ex_h, den_h, h_h, out_h,
              dnv, pkb, exb, coefv, z64, dstb, rowbuf, sacc,
              semg, sems):
    c = lax.axis_index("c")
    s = lax.axis_index("s")
    zero = jnp.zeros((L,), jnp.float32)
    izero = jnp.zeros((L,), jnp.int32)

    pltpu.sync_copy(den_h.at[c], dnv)

    def zrow(r, carry):
        for b in range(NB):
            for q in range(QR):
                rowbuf[b, r, pl.ds(q * L, L)] = zero
        return carry
    lax.fori_loop(0, CK, zrow, 0)
    zoff = (c * NS + s) * CK
    for k in range(CK // L):
        z64[pl.ds(k * L, L)] = lax.iota(jnp.int32, L) + (zoff + k * L)

    base = s * RPT
    nfull = RPT // CK
    rem = RPT - nfull * CK
    for w in range(nfull):
        pltpu.sync_copy(rowbuf.at[0], sacc.at[pl.ds(base + w * CK, CK)])
    pltpu.sync_copy(rowbuf.at[0, pl.ds(0, rem)],
                    sacc.at[pl.ds(base + nfull * CK, rem)])

    @pl.when(s == 0)
    def _():
        pltpu.sync_copy(rowbuf.at[0, pl.ds(0, REM)],
                        sacc.at[pl.ds(RPT * NS, REM)])

    plsc.subcore_barrier()

    del z64

    def scale_rows(ww):
        def rowfn(r, rcarry):
            for i in range(2):
                rv = izero + (2 * r + i)
                cs = plsc.load_gather(coefv, [rv])
                for q in range(QR):
                    ql = pl.ds(q * L, L)
                    ww[2 * r + i, ql] = ww[2 * r + i, ql] * cs
            return rcarry
        lax.fori_loop(0, CK // 2, rowfn, 0)

    def p2(g, carry):
        cbase = s * NCH + c * NCH2 + g * G
        pltpu.sync_copy(pk_h.at[pl.ds(cbase, G)], pkb)
        pltpu.sync_copy(ex_h.at[pl.ds(cbase * CK, G * CK)], exb)
        cpg = [None] * G
        cpg[0] = pltpu.async_copy(h_h.at[pkb.at[0, 0]], rowbuf.at[0], semg[0])
        for t in range(G):
            b = t % NB
            if t < G - 1:
                bn = (t + 1) % NB
                cpg[t + 1] = pltpu.async_copy(h_h.at[pkb.at[t + 1, 0]],
                                              rowbuf.at[bn], semg[bn])
            cpg[t].wait()
            for k in range(CK // L):
                sl = pl.ds(k * L, L)
                dv = pkb[t, 1, sl]
                dstb[b, sl] = dv
                dn = plsc.load_gather(dnv, [dv])
                ex = exb[pl.ds(t * CK + k * L, L)]
                coefv[sl] = ex / (dn + 1e-16)
            scale_rows(rowbuf.at[b])
            pltpu.sync_copy(rowbuf.at[b], sacc.at[dstb.at[b]], add=True)
        return carry
    lax.fori_loop(0, NG2, p2, 0)

    plsc.subcore_barrier()

    for w in range(nfull):
        pltpu.sync_copy(sacc.at[pl.ds(base + w * CK, CK)], rowbuf.at[0])
        pltpu.sync_copy(rowbuf.at[0], out_h.at[c, pl.ds(base + w * CK, CK)])
    pltpu.sync_copy(sacc.at[pl.ds(base + nfull * CK, rem)],
                    rowbuf.at[0, pl.ds(0, rem)])
    pltpu.sync_copy(rowbuf.at[0, pl.ds(0, rem)],
                    out_h.at[c, pl.ds(base + nfull * CK, rem)])

    @pl.when(s == 0)
    def _():
        pltpu.sync_copy(sacc.at[pl.ds(RPT * NS, REM)],
                        rowbuf.at[1, pl.ds(0, REM)])
        pltpu.sync_copy(rowbuf.at[1, pl.ds(0, REM)],
                        out_h.at[c, pl.ds(RPT * NS, REM)])


def _scb_wrapper(pk_h, ex_h, den_h, h_h, out_h,
                 dnv, pkb, exb, coefv, z64, dstb, rowbuf, sacc,
                 sg0, sg1, sg2, sg3, ss0, ss1, ss2, ss3):
    _scb_body(pk_h, ex_h, den_h, h_h, out_h,
              dnv, pkb, exb, coefv, z64, dstb, rowbuf, sacc,
              (sg0, sg1, sg2, sg3), (ss0, ss1, ss2, ss3))


_scb = pl.kernel(
    _scb_wrapper,
    out_type=jax.ShapeDtypeStruct((NC, N, H), jnp.float32),
    mesh=_MESH,
    compiler_params=_SC_PARAMS,
    scratch_types=[
        pltpu.VMEM((N,), jnp.float32),
        pltpu.VMEM((G, 3, CK), jnp.int32),
        pltpu.VMEM((G * CK,), jnp.float32),
        pltpu.VMEM((CK,), jnp.float32),
        pltpu.VMEM((CK,), jnp.int32),
        pltpu.VMEM((NB, CK), jnp.int32),
        pltpu.VMEM((NB, CK, H), jnp.float32),
        pltpu.VMEM_SHARED((N, H), jnp.float32),
        pltpu.SemaphoreType.DMA,
        pltpu.SemaphoreType.DMA,
        pltpu.SemaphoreType.DMA,
        pltpu.SemaphoreType.DMA,
        pltpu.SemaphoreType.DMA,
        pltpu.SemaphoreType.DMA,
        pltpu.SemaphoreType.DMA,
        pltpu.SemaphoreType.DMA,
    ],
)


def _post_body(p_ref, b_ref, wl_ref, bl_ref, o_ref):
    t = p_ref[0] + p_ref[1] + b_ref[...]
    t = jnp.maximum(t, 0.0)
    o_ref[...] = (jnp.dot(t, wl_ref[...], preferred_element_type=jnp.float32)
                  + bl_ref[...])


_post = pl.pallas_call(
    _post_body,
    out_shape=jax.ShapeDtypeStruct((N, 1), jnp.float32),
)


def kernel(node_static_features, edge_static_features, edge_index, W,
           att_src, att_dst, W_edge, att_edge, bias, W_lin, b_lin):
    x = node_static_features.astype(jnp.float32)
    ea3 = edge_static_features.astype(jnp.float32).reshape(E // EPR, EPR * DE)
    h, a_s, a_d, ae2, sh = _pre(
        x, ea3, W, att_src.reshape(H, 1), att_dst.reshape(H, 1),
        W_edge, att_edge.reshape(H, 1))
    pad = jnp.zeros((EP - E,), jnp.int32)
    src_p = jnp.concatenate([edge_index[:, 0], pad]).reshape(EP // CK, 1, CK)
    dst_p = jnp.concatenate([edge_index[:, 1], pad]).reshape(EP // CK, 1, CK)
    ae_i = lax.bitcast_convert_type(
        jnp.concatenate([ae2.reshape(E), jnp.full((EP - E,), -1e30,
                                                  jnp.float32)]), jnp.int32)
    pk = jnp.concatenate([src_p, dst_p, ae_i.reshape(EP // CK, 1, CK)], axis=1)
    sh16 = jnp.broadcast_to(sh.reshape(()), (L,))
    ex, den = _sca(pk, a_s.reshape(N), a_d.reshape(N), sh16)
    parts = _scb(pk, ex, den, h)
    return _post(parts, bias.reshape(1, H), W_lin, b_lin.reshape(1, 1))


# speedup vs baseline: 1.7070x; 1.7070x over previous
import jax
import jax.numpy as jnp
from jax import lax
from jax.experimental import pallas as pl
from jax.experimental.pallas import tpu as pltpu
from jax.experimental.pallas import tpu_sc as plsc

N = 10000
E = 320000
D = 128
DE = 16
H = 128

NC = 2
NS = 16
L = 16

CK = 64
NCH = 320
NCH2 = NCH // NC
EP = CK * NCH * NS
G = 8
NG1 = NCH // G
NG2 = NCH2 // G
NB = 4
RPT = 624
REM = N - RPT * NS
EPR = 128
QR = H // L

_SC_PARAMS = pltpu.CompilerParams(needs_layout_passes=False,
                                  use_tc_tiling_on_sc=False)
_MESH = plsc.VectorSubcoreMesh(core_axis_name="c", subcore_axis_name="s")


def _pre_body(x_ref, ea_ref, w_ref, asr_ref, adr_ref, wer_ref, aer_ref,
              h_ref, as_ref, ad_ref, ae_ref, sh_ref):
    h = jnp.dot(x_ref[...], w_ref[...], preferred_element_type=jnp.float32)
    h_ref[...] = h
    a_s = jnp.dot(h, asr_ref[...], preferred_element_type=jnp.float32)
    a_d = jnp.dot(h, adr_ref[...], preferred_element_type=jnp.float32)
    as_ref[...] = a_s
    ad_ref[...] = a_d
    u = jnp.dot(wer_ref[...], aer_ref[...], preferred_element_type=jnp.float32)
    urep = jnp.concatenate([u] * EPR, axis=0)
    row = lax.broadcasted_iota(jnp.int32, (DE * EPR, EPR), 0)
    col = lax.broadcasted_iota(jnp.int32, (DE * EPR, EPR), 1)
    u3 = jnp.where((row // DE) == col, urep, 0.0)
    ae = jnp.dot(ea_ref[...], u3, preferred_element_type=jnp.float32)
    ae_ref[...] = ae
    sh = jnp.maximum(jnp.max(a_s) + jnp.max(a_d) + jnp.max(ae), 0.0)
    sh_ref[...] = jnp.zeros((1, 1), jnp.float32) + sh


_pre = pl.pallas_call(
    _pre_body,
    out_shape=[
        jax.ShapeDtypeStruct((N, H), jnp.float32),
        jax.ShapeDtypeStruct((N, 1), jnp.float32),
        jax.ShapeDtypeStruct((N, 1), jnp.float32),
        jax.ShapeDtypeStruct((E // EPR, EPR), jnp.float32),
        jax.ShapeDtypeStruct((1, 1), jnp.float32),
    ],
)


def _sca_body(pk_h, as_h, ad_h, sh_h, ex_h, den_h,
              asv, adv, dnv, shv, pkb, exg, dstg, exo, sden_a,
              semsa, semsb):
    c = lax.axis_index("c")
    s = lax.axis_index("s")
    zero = jnp.zeros((L,), jnp.float32)
    izero = jnp.zeros((L,), jnp.int32)

    pltpu.sync_copy(as_h, asv)
    pltpu.sync_copy(ad_h, adv)
    pltpu.sync_copy(sh_h, shv)

    for k in range(2 * CK // L):
        sl = pl.ds(k * L, L)
        for p in range(2):
            exg[p, sl] = zero
            dstg[p, sl] = (lax.iota(jnp.int32, L)
                           + (s * 4 * CK + p * 2 * CK + k * L))

    def zden(i, carry):
        dnv[pl.ds(i * L, L)] = zero
        return carry
    lax.fori_loop(0, N // L, zden, 0)

    @pl.when(s == 0)
    def _():
        pltpu.sync_copy(dnv, sden_a)

    plsc.subcore_barrier()

    shift = shv[...]

    pltpu.async_copy(exg.at[0], sden_a.at[dstg.at[0]], semsa, add=True)
    pltpu.async_copy(exg.at[1], sden_a.at[dstg.at[1]], semsb, add=True)

    def p1(g, carry):
        pltpu.sync_copy(pk_h.at[pl.ds(s * NCH + g * G, G)], pkb)
        for pair in range(G // 2):
            p = pair % 2
            sem = semsa if p == 0 else semsb
            pltpu.make_async_copy(exg.at[p], sden_a.at[dstg.at[p]], sem).wait()
            for u in range(2):
                tc = pair * 2 + u
                for k in range(CK // L):
                    sl = pl.ds(k * L, L)
                    sv = pkb[tc, 0, sl]
                    dv = pkb[tc, 1, sl]
                    ae = plsc.bitcast(pkb[tc, 2, sl], jnp.float32)
                    av = plsc.load_gather(asv, [sv])
                    bv = plsc.load_gather(adv, [dv])
                    al = av + bv + ae
                    al = jnp.where(al >= 0.0, al, al * 0.2)
                    ex = jnp.exp(al - shift)
                    exg[p, pl.ds(u * CK + k * L, L)] = ex
                    dstg[p, pl.ds(u * CK + k * L, L)] = dv
                    exo[pl.ds(tc * CK + k * L, L)] = ex
            pltpu.async_copy(exg.at[p], sden_a.at[dstg.at[p]], sem, add=True)
        pltpu.sync_copy(exo, ex_h.at[pl.ds((s * NCH + g * G) * CK, G * CK)])
        return carry
    lax.fori_loop(0, NG1, p1, 0)
    pltpu.make_async_copy(exg.at[0], sden_a.at[dstg.at[0]], semsa).wait()
    pltpu.make_async_copy(exg.at[1], sden_a.at[dstg.at[1]], semsb).wait()

    plsc.subcore_barrier()

    base = s * RPT
    pltpu.sync_copy(sden_a.at[pl.ds(base, RPT)], dnv.at[pl.ds(0, RPT)])
    pltpu.sync_copy(dnv.at[pl.ds(0, RPT)], den_h.at[c, pl.ds(base, RPT)])

    @pl.when(s == 0)
    def _():
        pltpu.sync_copy(sden_a.at[pl.ds(RPT * NS, REM)],
                        dnv.at[pl.ds(RPT, REM)])
        pltpu.sync_copy(dnv.at[pl.ds(RPT, REM)],
                        den_h.at[c, pl.ds(RPT * NS, REM)])


_sca = pl.kernel(
    _sca_body,
    out_type=[
        jax.ShapeDtypeStruct((EP,), jnp.float32),
        jax.ShapeDtypeStruct((NC, N), jnp.float32),
    ],
    mesh=_MESH,
    compiler_params=_SC_PARAMS,
    scratch_types=[
        pltpu.VMEM((N,), jnp.float32),
        pltpu.VMEM((N,), jnp.float32),
        pltpu.VMEM((N,), jnp.float32),
        pltpu.VMEM((L,), jnp.float32),
        pltpu.VMEM((G, 3, CK), jnp.int32),
        pltpu.VMEM((2, 2 * CK), jnp.float32),
        pltpu.VMEM((2, 2 * CK), jnp.int32),
        pltpu.VMEM((G * CK,), jnp.float32),
        pltpu.VMEM_SHARED((N,), jnp.float32),
        pltpu.SemaphoreType.DMA,
        pltpu.SemaphoreType.DMA,
    ],
)


def _scb_body(pk_h, ex_h, den_h, h_h, out_h,
              dnv, pkb, exb, coefv, z64, dstb, rowbuf, sacc,
              semg, sems):
    c = lax.axis_index("c")
    s = lax.axis_index("s")
    zero = jnp.zeros((L,), jnp.float32)
    izero = jnp.zeros((L,), jnp.int32)

    pltpu.sync_copy(den_h.at[c], dnv)

    def zrow(r, carry):
        for b in range(NB):
            for q in range(QR):
                rowbuf[b, r, pl.ds(q * L, L)] = zero
        return carry
    lax.fori_loop(0, CK, zrow, 0)
    zoff = (c * NS + s) * CK
    for k in range(CK // L):
        z64[pl.ds(k * L, L)] = lax.iota(jnp.int32, L) + (zoff + k * L)

    base = s * RPT
    nfull = RPT // CK
    rem = RPT - nfull * CK
    for w in range(nfull):
        pltpu.sync_copy(rowbuf.at[0], sacc.at[pl.ds(base + w * CK, CK)])
    pltpu.sync_copy(rowbuf.at[0, pl.ds(0, rem)],
                    sacc.at[pl.ds(base + nfull * CK, rem)])

    @pl.when(s == 0)
    def _():
        pltpu.sync_copy(rowbuf.at[0, pl.ds(0, REM)],
                        sacc.at[pl.ds(RPT * NS, REM)])

    plsc.subcore_barrier()

    del z64

    def scale_rows(ww):
        def rowfn(r, rcarry):
            for i in range(2):
                rv = izero + (2 * r + i)
                cs = plsc.load_gather(coefv, [rv])
                for q in range(QR):
                    ql = pl.ds(q * L, L)
                    ww[2 * r + i, ql] = ww[2 * r + i, ql] * cs
            return rcarry
        lax.fori_loop(0, CK // 2, rowfn, 0)

    def p2(g, carry):
        cbase = s * NCH + c * NCH2 + g * G
        pltpu.sync_copy(pk_h.at[pl.ds(cbase, G)], pkb)
        pltpu.sync_copy(ex_h.at[pl.ds(cbase * CK, G * CK)], exb)
        cpg = [None] * G
        cpg[0] = pltpu.async_copy(h_h.at[pkb.at[0, 0]], rowbuf.at[0], semg[0])
        for t in range(G):
            b = t % NB
            if t < G - 1:
                bn = (t + 1) % NB
                cpg[t + 1] = pltpu.async_copy(h_h.at[pkb.at[t + 1, 0]],
                                              rowbuf.at[bn], semg[bn])
            cpg[t].wait()
            for k in range(CK // L):
                sl = pl.ds(k * L, L)
                dv = pkb[t, 1, sl]
                dstb[b, sl] = dv
                dn = plsc.load_gather(dnv, [dv])
                ex = exb[pl.ds(t * CK + k * L, L)]
                coefv[sl] = ex / (dn + 1e-16)
            scale_rows(rowbuf.at[b])
            pltpu.sync_copy(rowbuf.at[b], sacc.at[dstb.at[b]], add=True)
        return carry
    lax.fori_loop(0, NG2, p2, 0)

    plsc.subcore_barrier()

    for w in range(nfull):
        pltpu.sync_copy(sacc.at[pl.ds(base + w * CK, CK)], rowbuf.at[0])
        pltpu.sync_copy(rowbuf.at[0], out_h.at[c, pl.ds(base + w * CK, CK)])
    pltpu.sync_copy(sacc.at[pl.ds(base + nfull * CK, rem)],
                    rowbuf.at[0, pl.ds(0, rem)])
    pltpu.sync_copy(rowbuf.at[0, pl.ds(0, rem)],
                    out_h.at[c, pl.ds(base + nfull * CK, rem)])

    @pl.when(s == 0)
    def _():
        pltpu.sync_copy(sacc.at[pl.ds(RPT * NS, REM)],
                        rowbuf.at[1, pl.ds(0, REM)])
        pltpu.sync_copy(rowbuf.at[1, pl.ds(0, REM)],
                        out_h.at[c, pl.ds(RPT * NS, REM)])


def _scb_wrapper(pk_h, ex_h, den_h, h_h, out_h,
                 dnv, pkb, exb, coefv, z64, dstb, rowbuf, sacc,
                 sg0, sg1, sg2, sg3, ss0, ss1, ss2, ss3):
    _scb_body(pk_h, ex_h, den_h, h_h, out_h,
              dnv, pkb, exb, coefv, z64, dstb, rowbuf, sacc,
              (sg0, sg1, sg2, sg3), (ss0, ss1, ss2, ss3))


_scb = pl.kernel(
    _scb_wrapper,
    out_type=jax.ShapeDtypeStruct((NC, N, H), jnp.float32),
    mesh=_MESH,
    compiler_params=_SC_PARAMS,
    scratch_types=[
        pltpu.VMEM((N,), jnp.float32),
        pltpu.VMEM((G, 3, CK), jnp.int32),
        pltpu.VMEM((G * CK,), jnp.float32),
        pltpu.VMEM((CK,), jnp.float32),
        pltpu.VMEM((CK,), jnp.int32),
        pltpu.VMEM((NB, CK), jnp.int32),
        pltpu.VMEM((NB, CK, H), jnp.float32),
        pltpu.VMEM_SHARED((N, H), jnp.float32),
        pltpu.SemaphoreType.DMA,
        pltpu.SemaphoreType.DMA,
        pltpu.SemaphoreType.DMA,
        pltpu.SemaphoreType.DMA,
        pltpu.SemaphoreType.DMA,
        pltpu.SemaphoreType.DMA,
        pltpu.SemaphoreType.DMA,
        pltpu.SemaphoreType.DMA,
    ],
)


def _post_body(p_ref, b_ref, wl_ref, bl_ref, o_ref):
    t = p_ref[0] + p_ref[1] + b_ref[...]
    t = jnp.maximum(t, 0.0)
    o_ref[...] = (jnp.dot(t, wl_ref[...], preferred_element_type=jnp.float32)
                  + bl_ref[...])


_post = pl.pallas_call(
    _post_body,
    out_shape=jax.ShapeDtypeStruct((N, 1), jnp.float32),
)


def kernel(node_static_features, edge_static_features, edge_index, W,
           att_src, att_dst, W_edge, att_edge, bias, W_lin, b_lin):
    x = node_static_features.astype(jnp.float32)
    ea3 = edge_static_features.astype(jnp.float32).reshape(E // EPR, EPR * DE)
    h, a_s, a_d, ae2, sh = _pre(
        x, ea3, W, att_src.reshape(H, 1), att_dst.reshape(H, 1),
        W_edge, att_edge.reshape(H, 1))
    pad = jnp.arange(EP - E, dtype=jnp.int32) % N
    src_p = jnp.concatenate([edge_index[:, 0], pad]).reshape(EP // CK, 1, CK)
    dst_p = jnp.concatenate([edge_index[:, 1], pad]).reshape(EP // CK, 1, CK)
    ae_i = lax.bitcast_convert_type(
        jnp.concatenate([ae2.reshape(E), jnp.full((EP - E,), -1e30,
                                                  jnp.float32)]), jnp.int32)
    pk = jnp.concatenate([src_p, dst_p, ae_i.reshape(EP // CK, 1, CK)], axis=1)
    sh16 = jnp.broadcast_to(sh.reshape(()), (L,))
    ex, den = _sca(pk, a_s.reshape(N), a_d.reshape(N), sh16)
    parts = _scb(pk, ex, den, h)
    return _post(parts, bias.reshape(1, H), W_lin, b_lin.reshape(1, 1))


# speedup vs baseline: 1.8201x; 1.0662x over previous
import jax
import jax.numpy as jnp
from jax import lax
from jax.experimental import pallas as pl
from jax.experimental.pallas import tpu as pltpu
from jax.experimental.pallas import tpu_sc as plsc

N = 10000
E = 320000
D = 128
DE = 16
H = 128

NC = 2
NS = 16
L = 16

CK = 64
NCH = 320
NCH2 = NCH // NC
EP = CK * NCH * NS
G = 8
NG1 = NCH // G
NG2 = NCH2 // G
NB = 4
RPT = 624
REM = N - RPT * NS
EPR = 128
QR = H // L

_SC_PARAMS = pltpu.CompilerParams(needs_layout_passes=False,
                                  use_tc_tiling_on_sc=False)
_MESH = plsc.VectorSubcoreMesh(core_axis_name="c", subcore_axis_name="s")


def _pre_body(x_ref, ea_ref, w_ref, asr_ref, adr_ref, wer_ref, aer_ref,
              h_ref, as_ref, ad_ref, ae_ref, sh_ref):
    h = jnp.dot(x_ref[...], w_ref[...], preferred_element_type=jnp.float32)
    h_ref[...] = h
    a_s = jnp.dot(h, asr_ref[...], preferred_element_type=jnp.float32)
    a_d = jnp.dot(h, adr_ref[...], preferred_element_type=jnp.float32)
    as_ref[...] = a_s
    ad_ref[...] = a_d
    u = jnp.dot(wer_ref[...], aer_ref[...], preferred_element_type=jnp.float32)
    urep = jnp.concatenate([u] * EPR, axis=0)
    row = lax.broadcasted_iota(jnp.int32, (DE * EPR, EPR), 0)
    col = lax.broadcasted_iota(jnp.int32, (DE * EPR, EPR), 1)
    u3 = jnp.where((row // DE) == col, urep, 0.0)
    ae = jnp.dot(ea_ref[...], u3, preferred_element_type=jnp.float32)
    ae_ref[...] = ae
    sh = jnp.maximum(jnp.max(a_s) + jnp.max(a_d) + jnp.max(ae), 0.0)
    sh_ref[...] = jnp.zeros((1, 1), jnp.float32) + sh


_pre = pl.pallas_call(
    _pre_body,
    out_shape=[
        jax.ShapeDtypeStruct((N, H), jnp.float32),
        jax.ShapeDtypeStruct((N, 1), jnp.float32),
        jax.ShapeDtypeStruct((N, 1), jnp.float32),
        jax.ShapeDtypeStruct((E // EPR, EPR), jnp.float32),
        jax.ShapeDtypeStruct((1, 1), jnp.float32),
    ],
)


def _sca_body(pk_h, as_h, ad_h, sh_h, ex_h, den_h,
              asv, adv, dnv, shv, pkb, exg, dstg, exo, sden_a,
              semsa, semsb):
    c = lax.axis_index("c")
    s = lax.axis_index("s")
    zero = jnp.zeros((L,), jnp.float32)
    izero = jnp.zeros((L,), jnp.int32)

    pltpu.sync_copy(as_h, asv)
    pltpu.sync_copy(ad_h, adv)
    pltpu.sync_copy(sh_h, shv)

    for k in range(2 * CK // L):
        sl = pl.ds(k * L, L)
        for p in range(2):
            exg[p, sl] = zero
            dstg[p, sl] = (lax.iota(jnp.int32, L)
                           + (s * 4 * CK + p * 2 * CK + k * L))

    def zden(i, carry):
        dnv[pl.ds(i * L, L)] = zero
        return carry
    lax.fori_loop(0, N // L, zden, 0)

    @pl.when(s == 0)
    def _():
        pltpu.sync_copy(dnv, sden_a)

    plsc.subcore_barrier()

    shift = shv[...]

    pltpu.async_copy(exg.at[0], sden_a.at[dstg.at[0]], semsa, add=True)
    pltpu.async_copy(exg.at[1], sden_a.at[dstg.at[1]], semsb, add=True)

    def p1(g, carry):
        pltpu.sync_copy(pk_h.at[pl.ds(s * NCH + g * G, G)], pkb)
        for pair in range(G // 2):
            p = pair % 2
            sem = semsa if p == 0 else semsb
            pltpu.make_async_copy(exg.at[p], sden_a.at[dstg.at[p]], sem).wait()
            for u in range(2):
                tc = pair * 2 + u
                for k in range(CK // L):
                    sl = pl.ds(k * L, L)
                    sv = pkb[tc, 0, sl]
                    dv = pkb[tc, 1, sl]
                    ae = plsc.bitcast(pkb[tc, 2, sl], jnp.float32)
                    av = plsc.load_gather(asv, [sv])
                    bv = plsc.load_gather(adv, [dv])
                    al = av + bv + ae
                    al = jnp.where(al >= 0.0, al, al * 0.2)
                    ex = jnp.exp(al - shift)
                    exg[p, pl.ds(u * CK + k * L, L)] = ex
                    dstg[p, pl.ds(u * CK + k * L, L)] = dv
                    exo[pl.ds(tc * CK + k * L, L)] = ex
            pltpu.async_copy(exg.at[p], sden_a.at[dstg.at[p]], sem, add=True)
        pltpu.sync_copy(exo, ex_h.at[pl.ds((s * NCH + g * G) * CK, G * CK)])
        return carry
    lax.fori_loop(0, NG1, p1, 0)
    pltpu.make_async_copy(exg.at[0], sden_a.at[dstg.at[0]], semsa).wait()
    pltpu.make_async_copy(exg.at[1], sden_a.at[dstg.at[1]], semsb).wait()

    plsc.subcore_barrier()

    base = s * RPT
    pltpu.sync_copy(sden_a.at[pl.ds(base, RPT)], dnv.at[pl.ds(0, RPT)])
    pltpu.sync_copy(dnv.at[pl.ds(0, RPT)], den_h.at[c, pl.ds(base, RPT)])

    @pl.when(s == 0)
    def _():
        pltpu.sync_copy(sden_a.at[pl.ds(RPT * NS, REM)],
                        dnv.at[pl.ds(RPT, REM)])
        pltpu.sync_copy(dnv.at[pl.ds(RPT, REM)],
                        den_h.at[c, pl.ds(RPT * NS, REM)])


_sca = pl.kernel(
    _sca_body,
    out_type=[
        jax.ShapeDtypeStruct((EP,), jnp.float32),
        jax.ShapeDtypeStruct((NC, N), jnp.float32),
    ],
    mesh=_MESH,
    compiler_params=_SC_PARAMS,
    scratch_types=[
        pltpu.VMEM((N,), jnp.float32),
        pltpu.VMEM((N,), jnp.float32),
        pltpu.VMEM((N,), jnp.float32),
        pltpu.VMEM((L,), jnp.float32),
        pltpu.VMEM((G, 3, CK), jnp.int32),
        pltpu.VMEM((2, 2 * CK), jnp.float32),
        pltpu.VMEM((2, 2 * CK), jnp.int32),
        pltpu.VMEM((G * CK,), jnp.float32),
        pltpu.VMEM_SHARED((N,), jnp.float32),
        pltpu.SemaphoreType.DMA,
        pltpu.SemaphoreType.DMA,
    ],
)


def _scb_body(pk_h, ex_h, den_h, h_h, out_h,
              dnv, pkb, exb, coefv, z64, dstb, rowbuf, sacc,
              semg, sems):
    c = lax.axis_index("c")
    s = lax.axis_index("s")
    zero = jnp.zeros((L,), jnp.float32)
    izero = jnp.zeros((L,), jnp.int32)

    pltpu.sync_copy(den_h.at[c], dnv)

    def zrow(r, carry):
        for b in range(NB):
            for q in range(QR):
                rowbuf[b, r, pl.ds(q * L, L)] = zero
        return carry
    lax.fori_loop(0, CK, zrow, 0)
    zoff = (c * NS + s) * CK
    for k in range(CK // L):
        z64[pl.ds(k * L, L)] = lax.iota(jnp.int32, L) + (zoff + k * L)

    base = s * RPT
    nfull = RPT // CK
    rem = RPT - nfull * CK
    for w in range(nfull):
        pltpu.sync_copy(rowbuf.at[0], sacc.at[pl.ds(base + w * CK, CK)])
    pltpu.sync_copy(rowbuf.at[0, pl.ds(0, rem)],
                    sacc.at[pl.ds(base + nfull * CK, rem)])

    @pl.when(s == 0)
    def _():
        pltpu.sync_copy(rowbuf.at[0, pl.ds(0, REM)],
                        sacc.at[pl.ds(RPT * NS, REM)])

    plsc.subcore_barrier()

    for b in range(1, NB):
        pltpu.async_copy(rowbuf.at[b], sacc.at[z64], sems[b], add=True)

    def scale_rows(ww):
        def rowfn(r, rcarry):
            for i in range(2):
                rv = izero + (2 * r + i)
                cs = plsc.load_gather(coefv, [rv])
                for q in range(QR):
                    ql = pl.ds(q * L, L)
                    ww[2 * r + i, ql] = ww[2 * r + i, ql] * cs
            return rcarry
        lax.fori_loop(0, CK // 2, rowfn, 0)

    def p2(g, carry):
        cbase = s * NCH + c * NCH2 + g * G
        pltpu.sync_copy(pk_h.at[pl.ds(cbase, G)], pkb)
        pltpu.sync_copy(ex_h.at[pl.ds(cbase * CK, G * CK)], exb)
        cpg = [None] * G
        cps = [None] * G
        cpg[0] = pltpu.async_copy(h_h.at[pkb.at[0, 0]], rowbuf.at[0], semg[0])
        for t in range(G):
            b = t % NB
            if t < 3:
                pltpu.make_async_copy(rowbuf.at[t + 1],
                                      sacc.at[pl.ds(0, CK)],
                                      sems[t + 1]).wait()
            else:
                cps[t - 3].wait()
            if t < G - 1:
                bn = (t + 1) % NB
                cpg[t + 1] = pltpu.async_copy(h_h.at[pkb.at[t + 1, 0]],
                                              rowbuf.at[bn], semg[bn])
            cpg[t].wait()
            for k in range(CK // L):
                sl = pl.ds(k * L, L)
                dv = pkb[t, 1, sl]
                dstb[b, sl] = dv
                dn = plsc.load_gather(dnv, [dv])
                ex = exb[pl.ds(t * CK + k * L, L)]
                coefv[sl] = ex / (dn + 1e-16)
            scale_rows(rowbuf.at[b])
            cps[t] = pltpu.async_copy(rowbuf.at[b], sacc.at[dstb.at[b]],
                                      sems[b], add=True)
        return carry
    lax.fori_loop(0, NG2, p2, 0)
    for b in range(1, NB):
        pltpu.make_async_copy(rowbuf.at[b], sacc.at[pl.ds(0, CK)],
                              sems[b]).wait()

    plsc.subcore_barrier()

    for w in range(nfull):
        pltpu.sync_copy(sacc.at[pl.ds(base + w * CK, CK)], rowbuf.at[0])
        pltpu.sync_copy(rowbuf.at[0], out_h.at[c, pl.ds(base + w * CK, CK)])
    pltpu.sync_copy(sacc.at[pl.ds(base + nfull * CK, rem)],
                    rowbuf.at[0, pl.ds(0, rem)])
    pltpu.sync_copy(rowbuf.at[0, pl.ds(0, rem)],
                    out_h.at[c, pl.ds(base + nfull * CK, rem)])

    @pl.when(s == 0)
    def _():
        pltpu.sync_copy(sacc.at[pl.ds(RPT * NS, REM)],
                        rowbuf.at[1, pl.ds(0, REM)])
        pltpu.sync_copy(rowbuf.at[1, pl.ds(0, REM)],
                        out_h.at[c, pl.ds(RPT * NS, REM)])


def _scb_wrapper(pk_h, ex_h, den_h, h_h, out_h,
                 dnv, pkb, exb, coefv, z64, dstb, rowbuf, sacc,
                 sg0, sg1, sg2, sg3, ss0, ss1, ss2, ss3):
    _scb_body(pk_h, ex_h, den_h, h_h, out_h,
              dnv, pkb, exb, coefv, z64, dstb, rowbuf, sacc,
              (sg0, sg1, sg2, sg3), (ss0, ss1, ss2, ss3))


_scb = pl.kernel(
    _scb_wrapper,
    out_type=jax.ShapeDtypeStruct((NC, N, H), jnp.float32),
    mesh=_MESH,
    compiler_params=_SC_PARAMS,
    scratch_types=[
        pltpu.VMEM((N,), jnp.float32),
        pltpu.VMEM((G, 3, CK), jnp.int32),
        pltpu.VMEM((G * CK,), jnp.float32),
        pltpu.VMEM((CK,), jnp.float32),
        pltpu.VMEM((CK,), jnp.int32),
        pltpu.VMEM((NB, CK), jnp.int32),
        pltpu.VMEM((NB, CK, H), jnp.float32),
        pltpu.VMEM_SHARED((N, H), jnp.float32),
        pltpu.SemaphoreType.DMA,
        pltpu.SemaphoreType.DMA,
        pltpu.SemaphoreType.DMA,
        pltpu.SemaphoreType.DMA,
        pltpu.SemaphoreType.DMA,
        pltpu.SemaphoreType.DMA,
        pltpu.SemaphoreType.DMA,
        pltpu.SemaphoreType.DMA,
    ],
)


def _post_body(p_ref, b_ref, wl_ref, bl_ref, o_ref):
    t = p_ref[0] + p_ref[1] + b_ref[...]
    t = jnp.maximum(t, 0.0)
    o_ref[...] = (jnp.dot(t, wl_ref[...], preferred_element_type=jnp.float32)
                  + bl_ref[...])


_post = pl.pallas_call(
    _post_body,
    out_shape=jax.ShapeDtypeStruct((N, 1), jnp.float32),
)


def kernel(node_static_features, edge_static_features, edge_index, W,
           att_src, att_dst, W_edge, att_edge, bias, W_lin, b_lin):
    x = node_static_features.astype(jnp.float32)
    ea3 = edge_static_features.astype(jnp.float32).reshape(E // EPR, EPR * DE)
    h, a_s, a_d, ae2, sh = _pre(
        x, ea3, W, att_src.reshape(H, 1), att_dst.reshape(H, 1),
        W_edge, att_edge.reshape(H, 1))
    pad = jnp.arange(EP - E, dtype=jnp.int32) % N
    src_p = jnp.concatenate([edge_index[:, 0], pad]).reshape(EP // CK, 1, CK)
    dst_p = jnp.concatenate([edge_index[:, 1], pad]).reshape(EP // CK, 1, CK)
    ae_i = lax.bitcast_convert_type(
        jnp.concatenate([ae2.reshape(E), jnp.full((EP - E,), -1e30,
                                                  jnp.float32)]), jnp.int32)
    pk = jnp.concatenate([src_p, dst_p, ae_i.reshape(EP // CK, 1, CK)], axis=1)
    sh16 = jnp.broadcast_to(sh.reshape(()), (L,))
    ex, den = _sca(pk, a_s.reshape(N), a_d.reshape(N), sh16)
    parts = _scb(pk, ex, den, h)
    return _post(parts, bias.reshape(1, H), W_lin, b_lin.reshape(1, 1))
